# Initial kernel scaffold; baseline (speedup 1.0000x reference)
import jax
import jax.numpy as jnp
from jax.experimental import pallas as pl


def kernel(indices, tables):
    raise NotImplementedError("write your pallas kernel here")


# speedup vs baseline: 6.2032x; 6.2032x over previous
import functools

import jax
import jax.numpy as jnp
from jax import lax
from jax.experimental import pallas as pl
from jax.experimental.pallas import tpu as pltpu
from jax.experimental.pallas import tpu_sc as plsc

F, B, L = 26, 4096, 20
VOCAB, DIM = 100000, 32
LANES = 16
NC, NS = 2, 16
NW = NC * NS

BAGS = F * B
BAGS_PER_W = BAGS // NW
CHUNK_BAGS = 64
CHUNK_IDS = CHUNK_BAGS * L
N_CHUNKS = BAGS_PER_W // CHUNK_BAGS
IDX_COLS = 128
IDX_ROWS = CHUNK_IDS // IDX_COLS
CHUNKS_PER_F = (B // CHUNK_BAGS)


def _sc_body(idx_hbm, tbl_hbm, out_hbm, raw_v, idx_v, rows_v, out_v, sem):
    wid = lax.axis_index("s") * NC + lax.axis_index("c")

    def chunk_body(k, _):
        g = wid * N_CHUNKS + k
        f = g // CHUNKS_PER_F
        offset = f * VOCAB

        pltpu.sync_copy(idx_hbm.at[pl.ds(g * CHUNK_IDS, CHUNK_IDS)], raw_v)

        off_vec = jnp.full((LANES,), offset, dtype=jnp.int32)
        vecs_per_row = IDX_COLS // LANES

        def remap_body(t, _):
            r = t // vecs_per_row
            c = (t % vecs_per_row) * LANES
            raw = raw_v[pl.ds(t * LANES, LANES)]
            idx_v[r, pl.ds(c, LANES)] = lax.rem(raw, VOCAB) + off_vec
            return 0

        lax.fori_loop(0, CHUNK_IDS // LANES, remap_body, 0)

        descs = [
            pltpu.async_copy(tbl_hbm.at[idx_v.at[j]],
                             rows_v.at[pl.ds(j * IDX_COLS, IDX_COLS)], sem)
            for j in range(IDX_ROWS)
        ]
        for d in descs:
            d.wait()

        def bag_body(b, _):
            base = b * L
            a0 = rows_v[base, pl.ds(0, LANES)]
            a1 = rows_v[base, pl.ds(LANES, LANES)]
            for l in range(1, L):
                a0 = a0 + rows_v[base + l, pl.ds(0, LANES)]
                a1 = a1 + rows_v[base + l, pl.ds(LANES, LANES)]
            out_v[b, pl.ds(0, LANES)] = a0
            out_v[b, pl.ds(LANES, LANES)] = a1
            return 0

        lax.fori_loop(0, CHUNK_BAGS, bag_body, 0)

        pltpu.sync_copy(out_v, out_hbm.at[pl.ds(g * CHUNK_BAGS, CHUNK_BAGS)])
        return 0

    lax.fori_loop(0, N_CHUNKS, chunk_body, 0)


@jax.jit
def kernel(indices, tables):
    idx_flat = indices.reshape(BAGS * L)
    tbl_flat = tables.reshape(F * VOCAB, DIM)

    mesh = plsc.VectorSubcoreMesh(core_axis_name="c", subcore_axis_name="s",
                                  num_cores=NC, num_subcores=NS)
    run = functools.partial(
        pl.kernel,
        out_type=jax.ShapeDtypeStruct((BAGS, DIM), jnp.float32),
        mesh=mesh,
        scratch_types=[
            pltpu.VMEM((CHUNK_IDS,), jnp.int32),
            pltpu.VMEM((IDX_ROWS, IDX_COLS), jnp.int32),
            pltpu.VMEM((CHUNK_IDS, DIM), jnp.float32),
            pltpu.VMEM((CHUNK_BAGS, DIM), jnp.float32),
            pltpu.SemaphoreType.DMA,
        ],
        compiler_params=pltpu.CompilerParams(use_tc_tiling_on_sc=False),
    )(_sc_body)
    out = run(idx_flat, tbl_flat)
    return out.reshape(F, B, DIM)
